# Initial kernel scaffold; baseline (speedup 1.0000x reference)
import jax
import jax.numpy as jnp
from jax.experimental import pallas as pl


def kernel(x, edge_index, W_self1, W_neigh1, b1, W_self2, W_neigh2, b2):
    raise NotImplementedError("write your pallas kernel here")


# speedup vs baseline: 3.2990x; 3.2990x over previous
import functools

import jax
import jax.numpy as jnp
from jax import lax
from jax.experimental import pallas as pl
from jax.experimental.pallas import tpu as pltpu
from jax.experimental.pallas import tpu_sc as plsc

N = 10000
E = 320000
D_IN = 128
D_HID = 128
D_CLS = 16
CHUNK = 64
NW = 32
EPAD = 327680
EPW = EPAD // NW
STEPS = EPW // CHUNK
NPIECE = 8
PSTEPS = STEPS // NPIECE
NPAD = 10240
RPS = NPAD // 16
NRM = 32
BM = 2560

_SC_PARAMS = pltpu.CompilerParams(needs_layout_passes=False)


def _sc_degree(dst3d, zeros_deg):
    mesh = plsc.VectorSubcoreMesh(core_axis_name="c", subcore_axis_name="s")

    @functools.partial(
        pl.kernel,
        out_type=jax.ShapeDtypeStruct((NW, NPAD), jnp.float32),
        mesh=mesh,
        scratch_types=[
            pltpu.VMEM((STEPS, CHUNK), jnp.int32),
            pltpu.VMEM((NPAD,), jnp.float32),
        ],
        compiler_params=_SC_PARAMS,
    )
    def k(dst_h, zerosd_h, deg_h, dst_v, deg_v):
        c = lax.axis_index("c")
        s = lax.axis_index("s")
        wid = c * 16 + s
        pltpu.sync_copy(dst_h.at[wid], dst_v)
        pltpu.sync_copy(zerosd_h, deg_v)
        ones = jnp.full((16,), 1.0, jnp.float32)

        def body(j, carry):
            for i in range(CHUNK // 16):
                vals = dst_v[j, pl.ds(i * 16, 16)]
                plsc.addupdate_scatter(deg_v, [vals], ones)
            return carry

        lax.fori_loop(0, STEPS, body, 0, unroll=False)
        pltpu.sync_copy(deg_v, deg_h.at[wid])

    return k(dst3d, zeros_deg)


def _sc_segment_mean(table, src3d, dst3d, zeros, degp):
    mesh = plsc.VectorSubcoreMesh(core_axis_name="c", subcore_axis_name="s")

    @functools.partial(
        pl.kernel,
        out_type=jax.ShapeDtypeStruct((2, NPAD, D_IN), jnp.float32),
        mesh=mesh,
        scratch_types=[
            pltpu.VMEM((PSTEPS, CHUNK), jnp.int32),
            pltpu.VMEM((PSTEPS, CHUNK), jnp.int32),
            pltpu.VMEM((PSTEPS, CHUNK), jnp.int32),
            pltpu.VMEM((PSTEPS, CHUNK), jnp.int32),
            pltpu.VMEM((CHUNK, D_IN), jnp.float32),
            pltpu.VMEM((CHUNK, D_IN), jnp.float32),
            pltpu.VMEM((NW, 128), jnp.float32),
            pltpu.VMEM((RPS,), jnp.float32),
            pltpu.VMEM((NRM, D_IN), jnp.float32),
            pltpu.VMEM_SHARED((NPAD, D_IN), jnp.float32),
            pltpu.SemaphoreType.DMA,
            pltpu.SemaphoreType.DMA,
            pltpu.SemaphoreType.DMA,
            pltpu.SemaphoreType.DMA,
        ],
        compiler_params=_SC_PARAMS,
    )
    def k(table_h, src_h, dst_h, zeros_h, degp_h, out_h,
          src0_v, src1_v, dst0_v, dst1_v, rows0_v, rows1_v,
          degc_v, rdeg_v, nrm_v, acc, sem0, sem1, semi_s, semi_d):
        c = lax.axis_index("c")
        s = lax.axis_index("s")
        wid = c * 16 + s
        base = s * RPS
        src_bufs = (src0_v, src1_v)
        dst_bufs = (dst0_v, dst1_v)
        pltpu.sync_copy(zeros_h, acc.at[pl.ds(base, RPS)])
        pltpu.sync_copy(src_h.at[wid, 0], src0_v)
        pltpu.sync_copy(dst_h.at[wid, 0], dst0_v)
        plsc.subcore_barrier()

        gcopy = lambda sbuf, j, rb, sm: pltpu.async_copy(
            table_h.at[sbuf.at[j]], rb, sm)
        gcopy(src0_v, 0, rows0_v, sem0)
        for p in range(NPIECE):
            pp = p % 2
            np_ = (p + 1) % 2
            if p + 1 < NPIECE:
                icp_s = pltpu.async_copy(src_h.at[wid, p + 1],
                                         src_bufs[np_], semi_s)
                icp_d = pltpu.async_copy(dst_h.at[wid, p + 1],
                                         dst_bufs[np_], semi_d)
            sp, dp = src_bufs[pp], dst_bufs[pp]

            def ibody(jj, carry, sp=sp, dp=dp):
                j = jj * 2
                pltpu.make_async_copy(table_h.at[sp.at[j]],
                                      rows0_v, sem0).wait()
                gcopy(sp, j + 1, rows1_v, sem1)
                pltpu.sync_copy(rows0_v, acc.at[dp.at[j]], add=True)
                pltpu.make_async_copy(table_h.at[sp.at[j + 1]],
                                      rows1_v, sem1).wait()
                gcopy(sp, j + 2, rows0_v, sem0)
                pltpu.sync_copy(rows1_v, acc.at[dp.at[j + 1]], add=True)
                return carry

            lax.fori_loop(0, PSTEPS // 2 - 1, ibody, 0, unroll=False)
            j = PSTEPS - 2
            pltpu.make_async_copy(table_h.at[sp.at[j]], rows0_v, sem0).wait()
            gcopy(sp, j + 1, rows1_v, sem1)
            pltpu.sync_copy(rows0_v, acc.at[dp.at[j]], add=True)
            pltpu.make_async_copy(table_h.at[sp.at[j + 1]],
                                  rows1_v, sem1).wait()
            if p + 1 < NPIECE:
                icp_s.wait()
                icp_d.wait()
                gcopy(src_bufs[np_], 0, rows0_v, sem0)
            pltpu.sync_copy(rows1_v, acc.at[dp.at[j + 1]], add=True)

        def rdeg_body(ch, carry):
            pltpu.sync_copy(degp_h.at[:, pl.ds(base + ch * 128, 128)], degc_v)
            for i in range(128 // 16):
                tot = jnp.zeros((16,), jnp.float32)
                for r in range(NW):
                    tot = tot + degc_v[r, pl.ds(i * 16, 16)]
                rdeg_v[pl.ds(ch * 128 + i * 16, 16)] = (
                    1.0 / jnp.maximum(tot, 1.0))
            return carry

        lax.fori_loop(0, RPS // 128, rdeg_body, 0, unroll=False)
        plsc.subcore_barrier()

        def norm_body(ch, carry):
            off = ch * NRM
            pltpu.sync_copy(acc.at[pl.ds(base + off, NRM)], nrm_v)
            for r16 in range(NRM // 16):
                rv = rdeg_v[pl.ds(off + r16 * 16, 16)]
                for l in range(16):
                    scale = jnp.full((16,), rv[l], jnp.float32)
                    for i in range(D_IN // 16):
                        sl = pl.ds(i * 16, 16)
                        nrm_v[r16 * 16 + l, sl] = nrm_v[r16 * 16 + l, sl] * scale
            pltpu.sync_copy(nrm_v, out_h.at[c, pl.ds(base + off, NRM)])
            return carry

        lax.fori_loop(0, RPS // NRM, norm_body, 0, unroll=False)

    return k(table, src3d, dst3d, zeros, degp)


def _tc_matmul(x, w):
    def body(x_ref, w_ref, o_ref):
        o_ref[...] = jnp.dot(x_ref[...], w_ref[...],
                             preferred_element_type=jnp.float32)

    return pl.pallas_call(
        body,
        grid=(NPAD // BM,),
        in_specs=[pl.BlockSpec((BM, D_IN), lambda i: (i, 0)),
                  pl.BlockSpec((D_IN, D_HID), lambda i: (0, 0))],
        out_specs=pl.BlockSpec((BM, D_HID), lambda i: (i, 0)),
        out_shape=jax.ShapeDtypeStruct((NPAD, D_HID), jnp.float32),
    )(x, w)


def _tc_mid(x, p1, w_self1, b1, w_self2, b2):
    def body(x_ref, p1_ref, ws1_ref, b1_ref, ws2_ref, b2_ref, h_ref, hsb_ref):
        hneigh = p1_ref[0] + p1_ref[1]
        h = (jnp.dot(x_ref[...], ws1_ref[...],
                     preferred_element_type=jnp.float32)
             + hneigh + b1_ref[...])
        h = jnp.maximum(h, 0.0)
        h_ref[...] = h
        hsb_ref[...] = jnp.dot(h, ws2_ref[...],
                               preferred_element_type=jnp.float32) + b2_ref[...]

    return pl.pallas_call(
        body,
        grid=(NPAD // BM,),
        in_specs=[pl.BlockSpec((BM, D_IN), lambda i: (i, 0)),
                  pl.BlockSpec((2, BM, D_IN), lambda i: (0, i, 0)),
                  pl.BlockSpec((D_IN, D_HID), lambda i: (0, 0)),
                  pl.BlockSpec((1, D_HID), lambda i: (0, 0)),
                  pl.BlockSpec((D_HID, D_CLS), lambda i: (0, 0)),
                  pl.BlockSpec((1, D_CLS), lambda i: (0, 0))],
        out_specs=[pl.BlockSpec((BM, D_HID), lambda i: (i, 0)),
                   pl.BlockSpec((BM, D_CLS), lambda i: (i, 0))],
        out_shape=[jax.ShapeDtypeStruct((NPAD, D_HID), jnp.float32),
                   jax.ShapeDtypeStruct((NPAD, D_CLS), jnp.float32)],
    )(x, p1, w_self1, b1, w_self2, b2)


def _tc_post(hsb, p2, w_neigh2):
    def body(hsb_ref, p2_ref, wn2_ref, o_ref):
        hn = p2_ref[0] + p2_ref[1]
        o_ref[...] = hsb_ref[...] + jnp.dot(
            hn, wn2_ref[...], preferred_element_type=jnp.float32)

    return pl.pallas_call(
        body,
        grid=(NPAD // BM,),
        in_specs=[pl.BlockSpec((BM, D_CLS), lambda i: (i, 0)),
                  pl.BlockSpec((2, BM, D_HID), lambda i: (0, i, 0)),
                  pl.BlockSpec((D_HID, D_CLS), lambda i: (0, 0))],
        out_specs=pl.BlockSpec((BM, D_CLS), lambda i: (i, 0)),
        out_shape=jax.ShapeDtypeStruct((NPAD, D_CLS), jnp.float32),
    )(hsb, p2, w_neigh2)


def kernel(x, edge_index, W_self1, W_neigh1, b1, W_self2, W_neigh2, b2):
    pad = EPAD - E
    src_i = jnp.concatenate(
        [edge_index[0].astype(jnp.int32), jnp.zeros((pad,), jnp.int32)])
    dst_i = jnp.concatenate(
        [edge_index[1].astype(jnp.int32), jnp.full((pad,), N, jnp.int32)])
    src4d = src_i.reshape(NW, NPIECE, PSTEPS, CHUNK)
    dst4d = dst_i.reshape(NW, NPIECE, PSTEPS, CHUNK)
    dst3d = dst_i.reshape(NW, STEPS, CHUNK)
    xp = jnp.pad(x, ((0, NPAD - N), (0, 0)))
    zeros = jnp.zeros((RPS, D_IN), jnp.float32)
    zeros_deg = jnp.zeros((NPAD,), jnp.float32)
    b1r = b1.reshape(1, D_HID)
    b2r = b2.reshape(1, D_CLS)

    degp = _sc_degree(dst3d, zeros_deg)
    z1 = _tc_matmul(xp, W_neigh1)
    p1 = _sc_segment_mean(z1, src4d, dst4d, zeros, degp)
    h, hsb = _tc_mid(xp, p1, W_self1, b1r, W_self2, b2r)
    p2 = _sc_segment_mean(h, src4d, dst4d, zeros, degp)
    out = _tc_post(hsb, p2, W_neigh2)
    return out[:N]
